# baseline (speedup 1.0000x reference)
import functools

import jax
import jax.numpy as jnp
from jax.experimental import pallas as pl
from jax.experimental.pallas import tpu as pltpu
from jax.experimental.pallas import tpu_sc as plsc

P, L, NL, NQ = 50000, 8, 10000, 30000
F32 = jnp.float32
BF16 = jnp.bfloat16

ZS = {'traffic': (1385.4059, 859.8119), 'packets': (1.4015, 0.8933),
      'eq_lambda': (1350.9712, 858.3162), 'avg_pkts_lambda': (0.9117, 0.9724),
      'exp_max_factor': (6.6636, 4.7151), 'pkts_lambda_on': (0.9116, 1.6513),
      'avg_t_off': (1.6649, 2.3564), 'avg_t_on': (1.6649, 2.3564),
      'ar_a': (0.0, 1.0), 'sigma': (0.0, 1.0),
      'capacity': (27611.0918, 20090.6211), 'queue_size': (30259.1055, 21410.0957)}


_W = 128


def _sc_gather(table, idx):
    m0 = idx.shape[0]
    d = table.shape[1]
    m = ((m0 + _W - 1) // _W) * _W
    if m != m0:
        idx = jnp.concatenate([idx, jnp.zeros((m - m0,), jnp.int32)])
    idx2 = idx.reshape(m // _W, _W)
    mesh = plsc.VectorSubcoreMesh(core_axis_name="core", subcore_axis_name="subcore")

    @functools.partial(
        pl.kernel,
        out_type=jax.ShapeDtypeStruct((m, d), table.dtype),
        mesh=mesh,
        compiler_params=pltpu.CompilerParams(use_tc_tiling_on_sc=False),
    )
    def gk(x_hbm, i_hbm, o_hbm):
        def body(i_vmem, o_vmem):
            pltpu.sync_copy(x_hbm.at[i_vmem.at[0]], o_vmem)

        pltpu.emit_pipeline(
            body,
            grid=(m // _W,),
            in_specs=[pl.BlockSpec((1, _W), lambda i: (i, 0))],
            out_specs=[pl.BlockSpec((_W, d), lambda i: (i, 0))],
            core_axis_name=("core", "subcore"),
            dimension_semantics=(pltpu.PARALLEL,),
        )(i_hbm, o_hbm)

    out = gk(table, idx2)
    return out[:m0] if m != m0 else out


def _dot(a, b):
    return jnp.dot(a, b, preferred_element_type=F32)


def _sig(x):
    return jax.nn.sigmoid(x)


def _mlp2_body(x_ref, w1, b1, w2, b2, o_ref):
    h = jax.nn.relu(_dot(x_ref[...], w1[...]) + b1[...])
    o_ref[...] = jax.nn.relu(_dot(h, w2[...]) + b2[...])


def _tc_mlp2(x, w1, b1, w2, b2, block):
    n, f = x.shape
    dh, do = w1.shape[1], w2.shape[1]
    full = lambda s: pl.BlockSpec(s, lambda i: (0, 0))
    return pl.pallas_call(
        _mlp2_body,
        grid=(n // block,),
        in_specs=[pl.BlockSpec((block, f), lambda i: (i, 0)),
                  full((f, dh)), full((1, dh)), full((dh, do)), full((1, do))],
        out_specs=pl.BlockSpec((block, do), lambda i: (i, 0)),
        out_shape=jax.ShapeDtypeStruct((n, do), F32),
    )(x, w1, b1.reshape(1, -1), w2, b2.reshape(1, -1))


def _link_embed_body(tr, cap, pol, w1, b1, w2, b2, wla, h_ref, zl_ref):
    tsum = jnp.sum(tr[:, :, 0:1], axis=1)
    load = tsum / cap[...]
    x = jnp.concatenate([load, pol[...]], axis=1)
    h = jax.nn.relu(_dot(x, w1[...]) + b1[...])
    h = jax.nn.relu(_dot(h, w2[...]) + b2[...])
    h_ref[...] = h
    zl_ref[...] = _dot(h, wla[...]).astype(BF16)


def _tc_link_embed(tr40, cap, pol, w1, b1, w2, b2, wla, block):
    full = lambda s: pl.BlockSpec(s, lambda i: (0, 0))
    return pl.pallas_call(
        _link_embed_body,
        grid=(NL // block,),
        in_specs=[pl.BlockSpec((block, 40, 128), lambda i: (i, 0, 0)),
                  pl.BlockSpec((block, 1), lambda i: (i, 0)),
                  pl.BlockSpec((block, 4), lambda i: (i, 0)),
                  full((5, 32)), full((1, 32)), full((32, 32)), full((1, 32)),
                  full((32, 128))],
        out_specs=[pl.BlockSpec((block, 32), lambda i: (i, 0)),
                   pl.BlockSpec((block, 128), lambda i: (i, 0))],
        out_shape=[jax.ShapeDtypeStruct((NL, 32), F32),
                   jax.ShapeDtypeStruct((NL, 128), BF16)],
    )(tr40, cap, pol, w1, b1.reshape(1, -1), w2, b2.reshape(1, -1), wla)


def _queue_embed_body(x_ref, w1, b1, w2, b2, wqa, ba, h_ref, zq_ref):
    h = jax.nn.relu(_dot(x_ref[...], w1[...]) + b1[...])
    h = jax.nn.relu(_dot(h, w2[...]) + b2[...])
    h_ref[...] = h
    zq_ref[...] = (_dot(h, wqa[...]) + ba[...]).astype(BF16)


def _tc_queue_embed(x, w1, b1, w2, b2, wqa, ba, block):
    full = lambda s: pl.BlockSpec(s, lambda i: (0, 0))
    return pl.pallas_call(
        _queue_embed_body,
        grid=(NQ // block,),
        in_specs=[pl.BlockSpec((block, 5), lambda i: (i, 0)),
                  full((5, 32)), full((1, 32)), full((32, 32)), full((1, 32)),
                  full((32, 128)), full((1, 128))],
        out_specs=[pl.BlockSpec((block, 32), lambda i: (i, 0)),
                   pl.BlockSpec((block, 128), lambda i: (i, 0))],
        out_shape=[jax.ShapeDtypeStruct((NQ, 32), F32),
                   jax.ShapeDtypeStruct((NQ, 128), BF16)],
    )(x, w1, b1.reshape(1, -1), w2, b2.reshape(1, -1), wqa, ba.reshape(1, -1))


def _bilstm_body(gq, gl, hf0, cf0, hb0, cb0, whf, whb,
                 ps_ref, hf_ref, cf_ref, hb_ref, cb_ref):
    z = gq[...].astype(F32) + gl[...].astype(F32)
    zf = z[:, :, :64]
    zb = z[:, :, 64:]
    hf = hf0[...]
    cf = cf0[...]
    sf = []
    for t in range(L):
        zt = zf[t] + _dot(hf, whf[...])
        i_, f_, g_, o_ = zt[:, :16], zt[:, 16:32], zt[:, 32:48], zt[:, 48:64]
        cf = _sig(f_) * cf + _sig(i_) * jnp.tanh(g_)
        hf = _sig(o_) * jnp.tanh(cf)
        sf.append(hf)
    hb = hb0[...]
    cb = cb0[...]
    sb = [None] * L
    for t in range(L - 1, -1, -1):
        zt = zb[t] + _dot(hb, whb[...])
        i_, f_, g_, o_ = zt[:, :16], zt[:, 16:32], zt[:, 32:48], zt[:, 48:64]
        cb = _sig(f_) * cb + _sig(i_) * jnp.tanh(g_)
        hb = _sig(o_) * jnp.tanh(cb)
        sb[t] = hb
    zpad = jnp.zeros((hf.shape[0], 96), F32)
    ps_ref[0] = jnp.concatenate([hf0[...], hb0[...], zpad], axis=1).astype(BF16)
    for t in range(L):
        ps_ref[t + 1] = jnp.concatenate([sf[t], sb[t], zpad], axis=1).astype(BF16)
    hf_ref[...] = hf
    cf_ref[...] = cf
    hb_ref[...] = hb
    cb_ref[...] = cb


def _tc_bilstm(gq, gl, hf, cf, hb, cb, whf, whb, block):
    full = lambda s: pl.BlockSpec(s, lambda i: (0, 0))
    st = lambda: pl.BlockSpec((block, 16), lambda i: (i, 0))
    return pl.pallas_call(
        _bilstm_body,
        grid=(P // block,),
        in_specs=[pl.BlockSpec((L, block, 128), lambda i: (0, i, 0)),
                  pl.BlockSpec((L, block, 128), lambda i: (0, i, 0)),
                  st(), st(), st(), st(),
                  full((16, 64)), full((16, 64))],
        out_specs=[pl.BlockSpec((L + 1, block, 128), lambda i: (0, i, 0)),
                   st(), st(), st(), st()],
        out_shape=[jax.ShapeDtypeStruct((L + 1, P, 128), BF16),
                   jax.ShapeDtypeStruct((P, 16), F32),
                   jax.ShapeDtypeStruct((P, 16), F32),
                   jax.ShapeDtypeStruct((P, 16), F32),
                   jax.ShapeDtypeStruct((P, 16), F32)],
    )(gq, gl, hf, cf, hb, cb, whf, whb)


def _queue_update_body(gps, h0, c0, wx, wh, b, wqa, ba, wzl, bzl,
                       h_ref, c_ref, zq_ref, zql_ref):
    g = gps[...]
    ps = g[0].astype(F32)
    for j in range(1, 16):
        ps = ps + g[j].astype(F32)
    ps = ps[:, :32]
    z = _dot(ps, wx[...]) + _dot(h0[...], wh[...]) + b[...]
    i_, f_, g_, o_ = z[:, :32], z[:, 32:64], z[:, 64:96], z[:, 96:128]
    c = _sig(f_) * c0[...] + _sig(i_) * jnp.tanh(g_)
    h = _sig(o_) * jnp.tanh(c)
    h_ref[...] = h
    c_ref[...] = c
    zq_ref[...] = (_dot(h, wqa[...]) + ba[...]).astype(BF16)
    zql_ref[...] = (_dot(h, wzl[...]) + bzl[...]).astype(BF16)


def _tc_queue_update(gps, h, c, wx, wh, b, wqa, ba, wzl, bzl, block):
    full = lambda s: pl.BlockSpec(s, lambda i: (0, 0))
    s32 = lambda: pl.BlockSpec((block, 32), lambda i: (i, 0))
    s128 = lambda: pl.BlockSpec((block, 128), lambda i: (i, 0))
    return pl.pallas_call(
        _queue_update_body,
        grid=(NQ // block,),
        in_specs=[pl.BlockSpec((16, block, 128), lambda i: (0, i, 0)),
                  s32(), s32(),
                  full((32, 128)), full((32, 128)), full((1, 128)),
                  full((32, 128)), full((1, 128)),
                  full((32, 128)), full((1, 128))],
        out_specs=[s32(), s32(), s128(), s128()],
        out_shape=[jax.ShapeDtypeStruct((NQ, 32), F32),
                   jax.ShapeDtypeStruct((NQ, 32), F32),
                   jax.ShapeDtypeStruct((NQ, 128), BF16),
                   jax.ShapeDtypeStruct((NQ, 128), BF16)],
    )(gps, h, c, wx, wh, b.reshape(1, -1), wqa, ba.reshape(1, -1),
      wzl, bzl.reshape(1, -1))


def _link_update_body(gql, h0, c0, wh, wla, h_ref, c_ref, zl_ref):
    g = gql[...]
    h = h0[...]
    c = c0[...]
    for t in range(3):
        z = g[t].astype(F32) + _dot(h, wh[...])
        i_, f_, g_, o_ = z[:, :32], z[:, 32:64], z[:, 64:96], z[:, 96:128]
        c = _sig(f_) * c + _sig(i_) * jnp.tanh(g_)
        h = _sig(o_) * jnp.tanh(c)
    h_ref[...] = h
    c_ref[...] = c
    zl_ref[...] = _dot(h, wla[...]).astype(BF16)


def _tc_link_update(gql, h, c, wh, wla, block):
    full = lambda s: pl.BlockSpec(s, lambda i: (0, 0))
    s32 = lambda: pl.BlockSpec((block, 32), lambda i: (i, 0))
    return pl.pallas_call(
        _link_update_body,
        grid=(NL // block,),
        in_specs=[pl.BlockSpec((3, block, 128), lambda i: (0, i, 0)),
                  s32(), s32(),
                  full((32, 128)), full((32, 128))],
        out_specs=[s32(), s32(),
                   pl.BlockSpec((block, 128), lambda i: (i, 0))],
        out_shape=[jax.ShapeDtypeStruct((NL, 32), F32),
                   jax.ShapeDtypeStruct((NL, 32), F32),
                   jax.ShapeDtypeStruct((NL, 128), BF16)],
    )(gql, h, c, wh, wla)


def _readout_body(ps, mcap, w1, b1, w2, b2, w3, b3, o_ref):
    acc = None
    for t in range(L):
        x = ps[t + 1][:, :32].astype(F32)
        r = jax.nn.relu(_dot(x, w1[...]) + b1[...])
        r = jax.nn.relu(_dot(r, w2[...]) + b2[...])
        ratio = _dot(r, w3[...]) + b3[...]
        term = ratio * mcap[t]
        acc = term if acc is None else acc + term
    o_ref[...] = acc


def _tc_readout(path_seq, mcap, w1, b1, w2, b2, w3, b3, block):
    full = lambda s: pl.BlockSpec(s, lambda i: (0, 0))
    return pl.pallas_call(
        _readout_body,
        grid=(P // block,),
        in_specs=[pl.BlockSpec((L + 1, block, 128), lambda i: (0, i, 0)),
                  pl.BlockSpec((L, block, 1), lambda i: (0, i, 0)),
                  full((32, 16)), full((1, 16)), full((16, 16)), full((1, 16)),
                  full((16, 1)), full((1, 1))],
        out_specs=pl.BlockSpec((block, 1), lambda i: (i, 0)),
        out_shape=jax.ShapeDtypeStruct((P, 1), F32),
    )(path_seq, mcap, w1, b1.reshape(1, -1), w2, b2.reshape(1, -1),
      w3, b3.reshape(1, -1))


@jax.jit
def _forward_impl(traffic, packets, eq_lambda, avg_pkts_lambda, exp_max_factor,
                  pkts_lambda_on, avg_t_off, avg_t_on, ar_a, sigma, capacity,
                  queue_size, weight, model, policy, priority, length,
                  queue_to_path, link_to_path, path_to_link, path_to_queue,
                  queue_to_link, pe_W1, pe_b1, pe_W2, pe_b2, le_W1, le_b1,
                  le_W2, le_b2, qe_W1, qe_b1, qe_W2, qe_b2, fw_Wx, fw_Wh,
                  fw_b, bw_Wx, bw_Wh, bw_b, qu_Wx, qu_Wh, qu_b, lu_Wx, lu_Wh,
                  lu_b, ro_W1, ro_b1, ro_W2, ro_b2, ro_W3, ro_b3):
    def zn(x, name):
        m, s = ZS[name]
        return (x - m) / s

    model_oh = jax.nn.one_hot(model, 7, dtype=F32)
    policy_oh = jax.nn.one_hot(policy, 4, dtype=F32)
    priority_oh = jax.nn.one_hot(priority, 3, dtype=F32)

    path_input = jnp.concatenate(
        [zn(traffic, 'traffic'), zn(packets, 'packets'), model_oh,
         zn(eq_lambda, 'eq_lambda'), zn(avg_pkts_lambda, 'avg_pkts_lambda'),
         zn(exp_max_factor, 'exp_max_factor'), zn(pkts_lambda_on, 'pkts_lambda_on'),
         zn(avg_t_off, 'avg_t_off'), zn(avg_t_on, 'avg_t_on'), ar_a, sigma],
        axis=1)
    queue_input = jnp.concatenate(
        [zn(queue_size, 'queue_size'), priority_oh, weight], axis=1)

    idx_q = queue_to_path.T.reshape(-1)
    idx_l = link_to_path.T.reshape(-1)
    idx_ps = (path_to_queue[:, :, 1] * P
              + path_to_queue[:, :, 0]).T.reshape(-1)
    idx_ql = queue_to_link.T.reshape(-1)
    idx_pl = path_to_link[:, :, 0].reshape(-1)

    wqa_p = jnp.concatenate([fw_Wx[:32], bw_Wx[:32]], axis=1)
    wla_p = jnp.concatenate([fw_Wx[32:], bw_Wx[32:]], axis=1)
    ba_p = jnp.concatenate([fw_b, bw_b])

    path_state = _tc_mlp2(path_input, pe_W1, pe_b1, pe_W2, pe_b2, 2000)
    h_fw = path_state[:, :16]
    h_bw = path_state[:, 16:]
    c_fw = jnp.zeros_like(h_fw)
    c_bw = jnp.zeros_like(h_bw)

    traffic128 = jnp.pad(traffic, ((0, 0), (0, 127)))
    capacity128 = jnp.pad(capacity, ((0, 0), (0, 127)))

    tr40 = _sc_gather(traffic128, idx_pl).reshape(NL, 40, 128)
    link_h, zl = _tc_link_embed(tr40, capacity, policy_oh, le_W1, le_b1,
                                le_W2, le_b2, wla_p, 400)
    queue_h, zq = _tc_queue_embed(queue_input, qe_W1, qe_b1, qe_W2, qe_b2,
                                  wqa_p, ba_p, 3000)
    queue_c = jnp.zeros_like(queue_h)
    link_c = jnp.zeros_like(link_h)

    cap_g = _sc_gather(capacity128, idx_l)[:, 0:1].reshape(L, P, 1)
    mask = (jnp.arange(L)[:, None] < length[None, :]).astype(F32)
    mcap = mask[:, :, None] / cap_g

    path_seq = None
    for _ in range(8):
        gq = _sc_gather(zq, idx_q).reshape(L, P, 128)
        gl = _sc_gather(zl, idx_l).reshape(L, P, 128)
        path_seq, h_fw, c_fw, h_bw, c_bw = _tc_bilstm(
            gq, gl, h_fw, c_fw, h_bw, c_bw, fw_Wh, bw_Wh, 1000)
        gps = _sc_gather(path_seq.reshape((L + 1) * P, 128),
                         idx_ps).reshape(16, NQ, 128)
        queue_h, queue_c, zq, zql = _tc_queue_update(
            gps, queue_h, queue_c, qu_Wx, qu_Wh, qu_b, wqa_p, ba_p,
            lu_Wx, lu_b, 1000)
        gql = _sc_gather(zql, idx_ql).reshape(3, NL, 128)
        link_h, link_c, zl = _tc_link_update(
            gql, link_h, link_c, lu_Wh, wla_p, 2000)

    return _tc_readout(path_seq, mcap, ro_W1, ro_b1, ro_W2, ro_b2,
                       ro_W3, ro_b3, 2000)


def kernel(*args):
    return _forward_impl(*args)


# speedup vs baseline: 1.5180x; 1.5180x over previous
import functools

import jax
import jax.numpy as jnp
from jax.experimental import pallas as pl
from jax.experimental.pallas import tpu as pltpu
from jax.experimental.pallas import tpu_sc as plsc

P, L, NL, NQ = 50000, 8, 10000, 30000
NQP = 30720
F32 = jnp.float32
BF16 = jnp.bfloat16

ZS = {'traffic': (1385.4059, 859.8119), 'packets': (1.4015, 0.8933),
      'eq_lambda': (1350.9712, 858.3162), 'avg_pkts_lambda': (0.9117, 0.9724),
      'exp_max_factor': (6.6636, 4.7151), 'pkts_lambda_on': (0.9116, 1.6513),
      'avg_t_off': (1.6649, 2.3564), 'avg_t_on': (1.6649, 2.3564),
      'ar_a': (0.0, 1.0), 'sigma': (0.0, 1.0),
      'capacity': (27611.0918, 20090.6211), 'queue_size': (30259.1055, 21410.0957)}


_W = 128


def _sc_gather(table, idx, pack=1):
    m0 = idx.shape[0]
    d = table.shape[1]
    m = ((m0 + _W - 1) // _W) * _W
    if pack > 1:
        assert m == m0
    if m != m0:
        idx = jnp.concatenate([idx, jnp.zeros((m - m0,), jnp.int32)])
    idx2 = idx.reshape(m // _W, _W)
    mesh = plsc.VectorSubcoreMesh(core_axis_name="core", subcore_axis_name="subcore")
    scratch = [pltpu.VMEM((_W, d), table.dtype)] if pack > 1 else []

    @functools.partial(
        pl.kernel,
        out_type=jax.ShapeDtypeStruct((m // pack, d), table.dtype),
        mesh=mesh,
        scratch_types=scratch,
        compiler_params=pltpu.CompilerParams(use_tc_tiling_on_sc=False,
                                             needs_layout_passes=False),
    )
    def gk(x_hbm, i_hbm, o_hbm, *maybe_buf):
        def body(i_vmem, o_vmem):
            if pack == 1:
                pltpu.sync_copy(x_hbm.at[i_vmem.at[0]], o_vmem)
            elif pack == 4:
                buf = maybe_buf[0]
                pltpu.sync_copy(x_hbm.at[i_vmem.at[0]], buf)

                @pl.loop(0, _W // 4)
                def _(k):
                    for c in range(4):
                        for h in range(2):
                            o_vmem[k, pl.ds(32 * c + 16 * h, 16)] = (
                                buf[4 * k + c, pl.ds(16 * h, 16)])
            else:
                buf = maybe_buf[0]
                pltpu.sync_copy(x_hbm.at[i_vmem.at[0]], buf)
                cols = jnp.zeros((16,), jnp.int32)
                for j in range(8):
                    rows = jax.lax.iota(jnp.int32, 16) + 16 * j
                    o_vmem[0, pl.ds(16 * j, 16)] = plsc.load_gather(
                        buf, [rows, cols])

        pltpu.emit_pipeline(
            body,
            grid=(m // _W,),
            in_specs=[pl.BlockSpec((1, _W), lambda i: (i, 0))],
            out_specs=[pl.BlockSpec((_W // pack, d), lambda i: (i, 0))],
            core_axis_name=("core", "subcore"),
            dimension_semantics=(pltpu.PARALLEL,),
        )(i_hbm, o_hbm)

    out = gk(table, idx2)
    return out[:m0] if m != m0 else out


def _dot(a, b):
    return jnp.dot(a, b, preferred_element_type=F32)


def _sig(x):
    return jax.nn.sigmoid(x)


def _mlp2_body(x_ref, w1, b1, w2, b2, o_ref):
    h = jax.nn.relu(_dot(x_ref[...], w1[...]) + b1[...])
    o_ref[...] = jax.nn.relu(_dot(h, w2[...]) + b2[...])


def _tc_mlp2(x, w1, b1, w2, b2, block):
    n, f = x.shape
    dh, do = w1.shape[1], w2.shape[1]
    full = lambda s: pl.BlockSpec(s, lambda i: (0, 0))
    return pl.pallas_call(
        _mlp2_body,
        grid=(n // block,),
        in_specs=[pl.BlockSpec((block, f), lambda i: (i, 0)),
                  full((f, dh)), full((1, dh)), full((dh, do)), full((1, do))],
        out_specs=pl.BlockSpec((block, do), lambda i: (i, 0)),
        out_shape=jax.ShapeDtypeStruct((n, do), F32),
    )(x, w1, b1.reshape(1, -1), w2, b2.reshape(1, -1))


def _link_embed_body(tr, cap, pol, w1, b1, w2, b2, wla, h_ref, zl_ref):
    tsum = jnp.sum(tr[...], axis=1, keepdims=True)
    load = tsum / cap[...]
    x = jnp.concatenate([load, pol[...]], axis=1)
    h = jax.nn.relu(_dot(x, w1[...]) + b1[...])
    h = jax.nn.relu(_dot(h, w2[...]) + b2[...])
    h_ref[...] = h
    zl_ref[...] = _dot(h, wla[...])


def _tc_link_embed(tr40, cap, pol, w1, b1, w2, b2, wla, block):
    full = lambda s: pl.BlockSpec(s, lambda i: (0, 0))
    return pl.pallas_call(
        _link_embed_body,
        grid=(NL // block,),
        in_specs=[pl.BlockSpec((block, 40), lambda i: (i, 0)),
                  pl.BlockSpec((block, 1), lambda i: (i, 0)),
                  pl.BlockSpec((block, 4), lambda i: (i, 0)),
                  full((5, 32)), full((1, 32)), full((32, 32)), full((1, 32)),
                  full((32, 128))],
        out_specs=[pl.BlockSpec((block, 32), lambda i: (i, 0)),
                   pl.BlockSpec((block, 128), lambda i: (i, 0))],
        out_shape=[jax.ShapeDtypeStruct((NL, 32), F32),
                   jax.ShapeDtypeStruct((NL, 128), F32)],
    )(tr40, cap, pol, w1, b1.reshape(1, -1), w2, b2.reshape(1, -1), wla)


def _queue_embed_body(x_ref, w1, b1, w2, b2, wqa, ba, h_ref, zq_ref):
    h = jax.nn.relu(_dot(x_ref[...], w1[...]) + b1[...])
    h = jax.nn.relu(_dot(h, w2[...]) + b2[...])
    h_ref[...] = h
    zq_ref[...] = _dot(h, wqa[...]) + ba[...]


def _tc_queue_embed(x, w1, b1, w2, b2, wqa, ba, block):
    full = lambda s: pl.BlockSpec(s, lambda i: (0, 0))
    return pl.pallas_call(
        _queue_embed_body,
        grid=(NQP // block,),
        in_specs=[pl.BlockSpec((block, 5), lambda i: (i, 0)),
                  full((5, 32)), full((1, 32)), full((32, 32)), full((1, 32)),
                  full((32, 128)), full((1, 128))],
        out_specs=[pl.BlockSpec((block, 32), lambda i: (i, 0)),
                   pl.BlockSpec((block, 128), lambda i: (i, 0))],
        out_shape=[jax.ShapeDtypeStruct((NQP, 32), F32),
                   jax.ShapeDtypeStruct((NQP, 128), F32)],
    )(x, w1, b1.reshape(1, -1), w2, b2.reshape(1, -1), wqa, ba.reshape(1, -1))


def _bilstm_body(gq, gl, hf0, cf0, hb0, cb0, whf, whb,
                 ps_ref, hf_ref, cf_ref, hb_ref, cb_ref):
    z = gq[...] + gl[...]
    zf = z[:, :, :64]
    zb = z[:, :, 64:]
    hf = hf0[...]
    cf = cf0[...]
    sf = []
    for t in range(L):
        zt = zf[t] + _dot(hf, whf[...])
        i_, f_, g_, o_ = zt[:, :16], zt[:, 16:32], zt[:, 32:48], zt[:, 48:64]
        cf = _sig(f_) * cf + _sig(i_) * jnp.tanh(g_)
        hf = _sig(o_) * jnp.tanh(cf)
        sf.append(hf)
    hb = hb0[...]
    cb = cb0[...]
    sb = [None] * L
    for t in range(L - 1, -1, -1):
        zt = zb[t] + _dot(hb, whb[...])
        i_, f_, g_, o_ = zt[:, :16], zt[:, 16:32], zt[:, 32:48], zt[:, 48:64]
        cb = _sig(f_) * cb + _sig(i_) * jnp.tanh(g_)
        hb = _sig(o_) * jnp.tanh(cb)
        sb[t] = hb
    zpad = jnp.zeros((hf.shape[0], 96), F32)
    ps_ref[0] = jnp.concatenate([hf0[...], hb0[...], zpad], axis=1)
    for t in range(L):
        ps_ref[t + 1] = jnp.concatenate([sf[t], sb[t], zpad], axis=1)
    hf_ref[...] = hf
    cf_ref[...] = cf
    hb_ref[...] = hb
    cb_ref[...] = cb


def _tc_bilstm(gq, gl, hf, cf, hb, cb, whf, whb, block):
    full = lambda s: pl.BlockSpec(s, lambda i: (0, 0))
    st = lambda: pl.BlockSpec((block, 16), lambda i: (i, 0))
    return pl.pallas_call(
        _bilstm_body,
        grid=(P // block,),
        in_specs=[pl.BlockSpec((L, block, 128), lambda i: (0, i, 0)),
                  pl.BlockSpec((L, block, 128), lambda i: (0, i, 0)),
                  st(), st(), st(), st(),
                  full((16, 64)), full((16, 64))],
        out_specs=[pl.BlockSpec((L + 1, block, 128), lambda i: (0, i, 0)),
                   st(), st(), st(), st()],
        out_shape=[jax.ShapeDtypeStruct((L + 1, P, 128), F32),
                   jax.ShapeDtypeStruct((P, 16), F32),
                   jax.ShapeDtypeStruct((P, 16), F32),
                   jax.ShapeDtypeStruct((P, 16), F32),
                   jax.ShapeDtypeStruct((P, 16), F32)],
    )(gq, gl, hf, cf, hb, cb, whf, whb)


def _queue_update_body(gps, h0, c0, wx, wh, b, wqa, ba, wzl, bzl,
                       h_ref, c_ref, zq_ref, zql_ref):
    g = gps[...]
    ps4 = g[0]
    for j in range(1, 16):
        ps4 = ps4 + g[j]
    for c in range(4):
        ps_c = ps4[:, 32 * c:32 * c + 32]
        z = _dot(ps_c, wx[...]) + _dot(h0[c], wh[...]) + b[...]
        i_, f_, g_, o_ = z[:, :32], z[:, 32:64], z[:, 64:96], z[:, 96:128]
        cn = _sig(f_) * c0[c] + _sig(i_) * jnp.tanh(g_)
        hn = _sig(o_) * jnp.tanh(cn)
        h_ref[c] = hn
        c_ref[c] = cn
        zq_ref[c] = _dot(hn, wqa[...]) + ba[...]
        zql_ref[c] = _dot(hn, wzl[...]) + bzl[...]


def _tc_queue_update(gps, h, c, wx, wh, b, wqa, ba, wzl, bzl, block4):
    full = lambda s: pl.BlockSpec(s, lambda i: (0, 0))
    nk = NQP // 4
    s32 = lambda: pl.BlockSpec((4, block4, 32), lambda i: (0, i, 0))
    s128 = lambda: pl.BlockSpec((4, block4, 128), lambda i: (0, i, 0))
    return pl.pallas_call(
        _queue_update_body,
        grid=(nk // block4,),
        in_specs=[pl.BlockSpec((16, block4, 128), lambda i: (0, i, 0)),
                  s32(), s32(),
                  full((32, 128)), full((32, 128)), full((1, 128)),
                  full((32, 128)), full((1, 128)),
                  full((32, 128)), full((1, 128))],
        out_specs=[s32(), s32(), s128(), s128()],
        out_shape=[jax.ShapeDtypeStruct((4, nk, 32), F32),
                   jax.ShapeDtypeStruct((4, nk, 32), F32),
                   jax.ShapeDtypeStruct((4, nk, 128), F32),
                   jax.ShapeDtypeStruct((4, nk, 128), F32)],
    )(gps, h, c, wx, wh, b.reshape(1, -1), wqa, ba.reshape(1, -1),
      wzl, bzl.reshape(1, -1))


def _link_update_body(gql, h0, c0, wh, wla, h_ref, c_ref, zl_ref):
    g = gql[...]
    h = h0[...]
    c = c0[...]
    for t in range(3):
        z = g[t] + _dot(h, wh[...])
        i_, f_, g_, o_ = z[:, :32], z[:, 32:64], z[:, 64:96], z[:, 96:128]
        c = _sig(f_) * c + _sig(i_) * jnp.tanh(g_)
        h = _sig(o_) * jnp.tanh(c)
    h_ref[...] = h
    c_ref[...] = c
    zl_ref[...] = _dot(h, wla[...])


def _tc_link_update(gql, h, c, wh, wla, block):
    full = lambda s: pl.BlockSpec(s, lambda i: (0, 0))
    s32 = lambda: pl.BlockSpec((block, 32), lambda i: (i, 0))
    return pl.pallas_call(
        _link_update_body,
        grid=(NL // block,),
        in_specs=[pl.BlockSpec((3, block, 128), lambda i: (0, i, 0)),
                  s32(), s32(),
                  full((32, 128)), full((32, 128))],
        out_specs=[s32(), s32(),
                   pl.BlockSpec((block, 128), lambda i: (i, 0))],
        out_shape=[jax.ShapeDtypeStruct((NL, 32), F32),
                   jax.ShapeDtypeStruct((NL, 32), F32),
                   jax.ShapeDtypeStruct((NL, 128), F32)],
    )(gql, h, c, wh, wla)


def _readout_body(ps, mcap, w1, b1, w2, b2, w3, b3, o_ref):
    acc = None
    for t in range(L):
        x = ps[t + 1][:, :32]
        r = jax.nn.relu(_dot(x, w1[...]) + b1[...])
        r = jax.nn.relu(_dot(r, w2[...]) + b2[...])
        ratio = _dot(r, w3[...]) + b3[...]
        term = ratio * mcap[t]
        acc = term if acc is None else acc + term
    o_ref[...] = acc


def _tc_readout(path_seq, mcap, w1, b1, w2, b2, w3, b3, block):
    full = lambda s: pl.BlockSpec(s, lambda i: (0, 0))
    return pl.pallas_call(
        _readout_body,
        grid=(P // block,),
        in_specs=[pl.BlockSpec((L + 1, block, 128), lambda i: (0, i, 0)),
                  pl.BlockSpec((L, block, 1), lambda i: (0, i, 0)),
                  full((32, 16)), full((1, 16)), full((16, 16)), full((1, 16)),
                  full((16, 1)), full((1, 1))],
        out_specs=pl.BlockSpec((block, 1), lambda i: (i, 0)),
        out_shape=jax.ShapeDtypeStruct((P, 1), F32),
    )(path_seq, mcap, w1, b1.reshape(1, -1), w2, b2.reshape(1, -1),
      w3, b3.reshape(1, -1))


@jax.jit
def _forward_impl(traffic, packets, eq_lambda, avg_pkts_lambda, exp_max_factor,
                  pkts_lambda_on, avg_t_off, avg_t_on, ar_a, sigma, capacity,
                  queue_size, weight, model, policy, priority, length,
                  queue_to_path, link_to_path, path_to_link, path_to_queue,
                  queue_to_link, pe_W1, pe_b1, pe_W2, pe_b2, le_W1, le_b1,
                  le_W2, le_b2, qe_W1, qe_b1, qe_W2, qe_b2, fw_Wx, fw_Wh,
                  fw_b, bw_Wx, bw_Wh, bw_b, qu_Wx, qu_Wh, qu_b, lu_Wx, lu_Wh,
                  lu_b, ro_W1, ro_b1, ro_W2, ro_b2, ro_W3, ro_b3):
    def zn(x, name):
        m, s = ZS[name]
        return (x - m) / s

    model_oh = jax.nn.one_hot(model, 7, dtype=F32)
    policy_oh = jax.nn.one_hot(policy, 4, dtype=F32)
    priority_oh = jax.nn.one_hot(priority, 3, dtype=F32)

    path_input = jnp.concatenate(
        [zn(traffic, 'traffic'), zn(packets, 'packets'), model_oh,
         zn(eq_lambda, 'eq_lambda'), zn(avg_pkts_lambda, 'avg_pkts_lambda'),
         zn(exp_max_factor, 'exp_max_factor'), zn(pkts_lambda_on, 'pkts_lambda_on'),
         zn(avg_t_off, 'avg_t_off'), zn(avg_t_on, 'avg_t_on'), ar_a, sigma],
        axis=1)
    queue_input = jnp.concatenate(
        [zn(queue_size, 'queue_size'), priority_oh, weight], axis=1)
    queue_input = jnp.pad(queue_input, ((0, NQP - NQ), (0, 0)))
    queue_input = queue_input.reshape(NQP // 4, 4, 5).transpose(1, 0, 2)
    queue_input = queue_input.reshape(NQP, 5)
    qperm = lambda q: (q % 4) * (NQP // 4) + q // 4

    idx_q = qperm(queue_to_path).T.reshape(-1)
    idx_l = link_to_path.T.reshape(-1)
    idx_ps = jnp.pad((path_to_queue[:, :, 1] * P
                      + path_to_queue[:, :, 0]).T,
                     ((0, 0), (0, NQP - NQ))).reshape(-1)
    idx_ql = qperm(queue_to_link).T.reshape(-1)
    idx_pl = path_to_link[:, :, 0].reshape(-1)

    wqa_p = jnp.concatenate([fw_Wx[:32], bw_Wx[:32]], axis=1)
    wla_p = jnp.concatenate([fw_Wx[32:], bw_Wx[32:]], axis=1)
    ba_p = jnp.concatenate([fw_b, bw_b])

    path_state = _tc_mlp2(path_input, pe_W1, pe_b1, pe_W2, pe_b2, 2000)
    h_fw = path_state[:, :16]
    h_bw = path_state[:, 16:]
    c_fw = jnp.zeros_like(h_fw)
    c_bw = jnp.zeros_like(h_bw)

    traffic128 = jnp.pad(traffic, ((0, 0), (0, 127)))
    capacity128 = jnp.pad(capacity, ((0, 0), (0, 127)))

    tr40 = _sc_gather(traffic128, idx_pl, pack=128).reshape(NL, 40)
    link_h, zl = _tc_link_embed(tr40, capacity, policy_oh, le_W1, le_b1,
                                le_W2, le_b2, wla_p, 400)
    queue_h, zq = _tc_queue_embed(queue_input, qe_W1, qe_b1, qe_W2, qe_b2,
                                  wqa_p, ba_p, 1024)
    queue_h = queue_h.reshape(4, NQP // 4, 32)
    zq = zq.reshape(NQP, 128)
    queue_c = jnp.zeros_like(queue_h)
    link_c = jnp.zeros_like(link_h)

    cap_g = _sc_gather(capacity128, idx_l, pack=128).reshape(L, P, 1)
    mask = (jnp.arange(L)[:, None] < length[None, :]).astype(F32)
    mcap = mask[:, :, None] / cap_g

    path_seq = None
    for _ in range(8):
        gq = _sc_gather(zq, idx_q).reshape(L, P, 128)
        gl = _sc_gather(zl, idx_l).reshape(L, P, 128)
        path_seq, h_fw, c_fw, h_bw, c_bw = _tc_bilstm(
            gq, gl, h_fw, c_fw, h_bw, c_bw, fw_Wh, bw_Wh, 1000)
        gps = _sc_gather(path_seq.reshape((L + 1) * P, 128),
                         idx_ps, pack=4).reshape(16, NQP // 4, 128)
        queue_h, queue_c, zq, zql = _tc_queue_update(
            gps, queue_h, queue_c, qu_Wx, qu_Wh, qu_b, wqa_p, ba_p,
            lu_Wx, lu_b, 768)
        zq = zq.reshape(NQP, 128)
        zql = zql.reshape(NQP, 128)
        gql = _sc_gather(zql, idx_ql).reshape(3, NL, 128)
        link_h, link_c, zl = _tc_link_update(
            gql, link_h, link_c, lu_Wh, wla_p, 2000)

    return _tc_readout(path_seq, mcap, ro_W1, ro_b1, ro_W2, ro_b2,
                       ro_W3, ro_b3, 2000)


def kernel(*args):
    return _forward_impl(*args)


# speedup vs baseline: 1.9101x; 1.2583x over previous
import functools

import jax
import jax.numpy as jnp
from jax.experimental import pallas as pl
from jax.experimental.pallas import tpu as pltpu
from jax.experimental.pallas import tpu_sc as plsc

P, L, NL, NQ = 50000, 8, 10000, 30000
NQP = 30720
F32 = jnp.float32
BF16 = jnp.bfloat16

ZS = {'traffic': (1385.4059, 859.8119), 'packets': (1.4015, 0.8933),
      'eq_lambda': (1350.9712, 858.3162), 'avg_pkts_lambda': (0.9117, 0.9724),
      'exp_max_factor': (6.6636, 4.7151), 'pkts_lambda_on': (0.9116, 1.6513),
      'avg_t_off': (1.6649, 2.3564), 'avg_t_on': (1.6649, 2.3564),
      'ar_a': (0.0, 1.0), 'sigma': (0.0, 1.0),
      'capacity': (27611.0918, 20090.6211), 'queue_size': (30259.1055, 21410.0957)}


_W = 128


def _sc_gather(table, idx, pack=1):
    m0 = idx.shape[0]
    d = table.shape[1]
    m = ((m0 + _W - 1) // _W) * _W
    if pack > 1:
        assert m == m0
    if m != m0:
        idx = jnp.concatenate([idx, jnp.zeros((m - m0,), jnp.int32)])
    idx2 = idx.reshape(m // _W, _W)
    mesh = plsc.VectorSubcoreMesh(core_axis_name="core", subcore_axis_name="subcore")
    scratch = [pltpu.VMEM((_W, d), table.dtype)] if pack > 1 else []

    @functools.partial(
        pl.kernel,
        out_type=jax.ShapeDtypeStruct((m // pack, d), table.dtype),
        mesh=mesh,
        scratch_types=scratch,
        compiler_params=pltpu.CompilerParams(use_tc_tiling_on_sc=False,
                                             needs_layout_passes=False),
    )
    def gk(x_hbm, i_hbm, o_hbm, *maybe_buf):
        def body(i_vmem, o_vmem):
            if pack == 1:
                pltpu.sync_copy(x_hbm.at[i_vmem.at[0]], o_vmem)
            elif pack == 4:
                buf = maybe_buf[0]
                pltpu.sync_copy(x_hbm.at[i_vmem.at[0]], buf)

                @pl.loop(0, _W // 4)
                def _(k):
                    for c in range(4):
                        for h in range(2):
                            o_vmem[k, pl.ds(32 * c + 16 * h, 16)] = (
                                buf[4 * k + c, pl.ds(16 * h, 16)])
            else:
                buf = maybe_buf[0]
                pltpu.sync_copy(x_hbm.at[i_vmem.at[0]], buf)
                cols = jnp.zeros((16,), jnp.int32)
                for j in range(8):
                    rows = jax.lax.iota(jnp.int32, 16) + 16 * j
                    o_vmem[0, pl.ds(16 * j, 16)] = plsc.load_gather(
                        buf, [rows, cols])

        pltpu.emit_pipeline(
            body,
            grid=(m // _W,),
            in_specs=[pl.BlockSpec((1, _W), lambda i: (i, 0))],
            out_specs=[pl.BlockSpec((_W // pack, d), lambda i: (i, 0))],
            core_axis_name=("core", "subcore"),
            dimension_semantics=(pltpu.PARALLEL,),
        )(i_hbm, o_hbm)

    out = gk(table, idx2)
    return out[:m0] if m != m0 else out


def _dot(a, b):
    return jnp.dot(a, b, preferred_element_type=F32)


def _sig(x):
    return jax.nn.sigmoid(x)


def _mlp2_body(x_ref, w1, b1, w2, b2, o_ref):
    h = jax.nn.relu(_dot(x_ref[...], w1[...]) + b1[...])
    o_ref[...] = jax.nn.relu(_dot(h, w2[...]) + b2[...])


def _tc_mlp2(x, w1, b1, w2, b2, block):
    n, f = x.shape
    dh, do = w1.shape[1], w2.shape[1]
    full = lambda s: pl.BlockSpec(s, lambda i: (0, 0))
    return pl.pallas_call(
        _mlp2_body,
        grid=(n // block,),
        in_specs=[pl.BlockSpec((block, f), lambda i: (i, 0)),
                  full((f, dh)), full((1, dh)), full((dh, do)), full((1, do))],
        out_specs=pl.BlockSpec((block, do), lambda i: (i, 0)),
        out_shape=jax.ShapeDtypeStruct((n, do), F32),
    )(x, w1, b1.reshape(1, -1), w2, b2.reshape(1, -1))


def _link_embed_body(tr, cap, pol, w1, b1, w2, b2, wla, h_ref, zl_ref):
    tsum = jnp.sum(tr[...], axis=1, keepdims=True)
    load = tsum / cap[...]
    x = jnp.concatenate([load, pol[...]], axis=1)
    h = jax.nn.relu(_dot(x, w1[...]) + b1[...])
    h = jax.nn.relu(_dot(h, w2[...]) + b2[...])
    h_ref[...] = h
    zl_ref[...] = _dot(h, wla[...])


def _tc_link_embed(tr40, cap, pol, w1, b1, w2, b2, wla, block):
    full = lambda s: pl.BlockSpec(s, lambda i: (0, 0))
    return pl.pallas_call(
        _link_embed_body,
        grid=(NL // block,),
        in_specs=[pl.BlockSpec((block, 40), lambda i: (i, 0)),
                  pl.BlockSpec((block, 1), lambda i: (i, 0)),
                  pl.BlockSpec((block, 4), lambda i: (i, 0)),
                  full((5, 32)), full((1, 32)), full((32, 32)), full((1, 32)),
                  full((32, 128))],
        out_specs=[pl.BlockSpec((block, 32), lambda i: (i, 0)),
                   pl.BlockSpec((block, 128), lambda i: (i, 0))],
        out_shape=[jax.ShapeDtypeStruct((NL, 32), F32),
                   jax.ShapeDtypeStruct((NL, 128), F32)],
    )(tr40, cap, pol, w1, b1.reshape(1, -1), w2, b2.reshape(1, -1), wla)


def _queue_embed_body(x_ref, w1, b1, w2, b2, wqa, ba, h_ref, zq_ref):
    h = jax.nn.relu(_dot(x_ref[...], w1[...]) + b1[...])
    h = jax.nn.relu(_dot(h, w2[...]) + b2[...])
    h_ref[...] = h
    zq_ref[...] = _dot(h, wqa[...]) + ba[...]


def _tc_queue_embed(x, w1, b1, w2, b2, wqa, ba, block):
    full = lambda s: pl.BlockSpec(s, lambda i: (0, 0))
    return pl.pallas_call(
        _queue_embed_body,
        grid=(NQ // block,),
        in_specs=[pl.BlockSpec((block, 5), lambda i: (i, 0)),
                  full((5, 32)), full((1, 32)), full((32, 32)), full((1, 32)),
                  full((32, 128)), full((1, 128))],
        out_specs=[pl.BlockSpec((block, 32), lambda i: (i, 0)),
                   pl.BlockSpec((block, 128), lambda i: (i, 0))],
        out_shape=[jax.ShapeDtypeStruct((NQ, 32), F32),
                   jax.ShapeDtypeStruct((NQ, 128), F32)],
    )(x, w1, b1.reshape(1, -1), w2, b2.reshape(1, -1), wqa, ba.reshape(1, -1))


def _bilstm_body(gq, gl, hf0, cf0, hb0, cb0, whf, whb,
                 ps_ref, hf_ref, cf_ref, hb_ref, cb_ref):
    z = gq[...] + gl[...]
    zf = z[:, :, :64]
    zb = z[:, :, 64:]
    hf = hf0[...]
    cf = cf0[...]
    sf = []
    for t in range(L):
        zt = zf[t] + _dot(hf, whf[...])
        i_, f_, g_, o_ = zt[:, :16], zt[:, 16:32], zt[:, 32:48], zt[:, 48:64]
        cf = _sig(f_) * cf + _sig(i_) * jnp.tanh(g_)
        hf = _sig(o_) * jnp.tanh(cf)
        sf.append(hf)
    hb = hb0[...]
    cb = cb0[...]
    sb = [None] * L
    for t in range(L - 1, -1, -1):
        zt = zb[t] + _dot(hb, whb[...])
        i_, f_, g_, o_ = zt[:, :16], zt[:, 16:32], zt[:, 32:48], zt[:, 48:64]
        cb = _sig(f_) * cb + _sig(i_) * jnp.tanh(g_)
        hb = _sig(o_) * jnp.tanh(cb)
        sb[t] = hb
    zpad = jnp.zeros((hf.shape[0], 96), F32)
    ps_ref[0] = jnp.concatenate([hf0[...], hb0[...], zpad], axis=1)
    for t in range(L):
        ps_ref[t + 1] = jnp.concatenate([sf[t], sb[t], zpad], axis=1)
    hf_ref[...] = hf
    cf_ref[...] = cf
    hb_ref[...] = hb
    cb_ref[...] = cb


def _tc_bilstm(gq, gl, hf, cf, hb, cb, whf, whb, block):
    full = lambda s: pl.BlockSpec(s, lambda i: (0, 0))
    st = lambda: pl.BlockSpec((block, 16), lambda i: (i, 0))
    return pl.pallas_call(
        _bilstm_body,
        grid=(P // block,),
        in_specs=[pl.BlockSpec((L, block, 128), lambda i: (0, i, 0)),
                  pl.BlockSpec((L, block, 128), lambda i: (0, i, 0)),
                  st(), st(), st(), st(),
                  full((16, 64)), full((16, 64))],
        out_specs=[pl.BlockSpec((L + 1, block, 128), lambda i: (0, i, 0)),
                   st(), st(), st(), st()],
        out_shape=[jax.ShapeDtypeStruct((L + 1, P, 128), F32),
                   jax.ShapeDtypeStruct((P, 16), F32),
                   jax.ShapeDtypeStruct((P, 16), F32),
                   jax.ShapeDtypeStruct((P, 16), F32),
                   jax.ShapeDtypeStruct((P, 16), F32)],
    )(gq, gl, hf, cf, hb, cb, whf, whb)


def _queue_update_body(gps, h0, c0, wx, wh, b, wqa, ba, wzl, bzl,
                       h_ref, c_ref, zq_ref, zql_ref):
    g = gps[...]
    ps = g[0]
    for j in range(1, 16):
        ps = ps + g[j]
    ps = ps[:, :32]
    z = _dot(ps, wx[...]) + _dot(h0[...], wh[...]) + b[...]
    i_, f_, g_, o_ = z[:, :32], z[:, 32:64], z[:, 64:96], z[:, 96:128]
    c = _sig(f_) * c0[...] + _sig(i_) * jnp.tanh(g_)
    h = _sig(o_) * jnp.tanh(c)
    h_ref[...] = h
    c_ref[...] = c
    zq_ref[...] = _dot(h, wqa[...]) + ba[...]
    zql_ref[...] = _dot(h, wzl[...]) + bzl[...]


def _tc_queue_update(gps, h, c, wx, wh, b, wqa, ba, wzl, bzl, block):
    full = lambda s: pl.BlockSpec(s, lambda i: (0, 0))
    s32 = lambda: pl.BlockSpec((block, 32), lambda i: (i, 0))
    s128 = lambda: pl.BlockSpec((block, 128), lambda i: (i, 0))
    return pl.pallas_call(
        _queue_update_body,
        grid=(NQ // block,),
        in_specs=[pl.BlockSpec((16, block, 128), lambda i: (0, i, 0)),
                  s32(), s32(),
                  full((32, 128)), full((32, 128)), full((1, 128)),
                  full((32, 128)), full((1, 128)),
                  full((32, 128)), full((1, 128))],
        out_specs=[s32(), s32(), s128(), s128()],
        out_shape=[jax.ShapeDtypeStruct((NQ, 32), F32),
                   jax.ShapeDtypeStruct((NQ, 32), F32),
                   jax.ShapeDtypeStruct((NQ, 128), F32),
                   jax.ShapeDtypeStruct((NQ, 128), F32)],
    )(gps, h, c, wx, wh, b.reshape(1, -1), wqa, ba.reshape(1, -1),
      wzl, bzl.reshape(1, -1))


def _link_update_body(gql, h0, c0, wh, wla, h_ref, c_ref, zl_ref):
    g = gql[...]
    h = h0[...]
    c = c0[...]
    for t in range(3):
        z = g[t] + _dot(h, wh[...])
        i_, f_, g_, o_ = z[:, :32], z[:, 32:64], z[:, 64:96], z[:, 96:128]
        c = _sig(f_) * c + _sig(i_) * jnp.tanh(g_)
        h = _sig(o_) * jnp.tanh(c)
    h_ref[...] = h
    c_ref[...] = c
    zl_ref[...] = _dot(h, wla[...])


def _tc_link_update(gql, h, c, wh, wla, block):
    full = lambda s: pl.BlockSpec(s, lambda i: (0, 0))
    s32 = lambda: pl.BlockSpec((block, 32), lambda i: (i, 0))
    return pl.pallas_call(
        _link_update_body,
        grid=(NL // block,),
        in_specs=[pl.BlockSpec((3, block, 128), lambda i: (0, i, 0)),
                  s32(), s32(),
                  full((32, 128)), full((32, 128))],
        out_specs=[s32(), s32(),
                   pl.BlockSpec((block, 128), lambda i: (i, 0))],
        out_shape=[jax.ShapeDtypeStruct((NL, 32), F32),
                   jax.ShapeDtypeStruct((NL, 32), F32),
                   jax.ShapeDtypeStruct((NL, 128), F32)],
    )(gql, h, c, wh, wla)


def _readout_body(ps, mcap, w1, b1, w2, b2, w3, b3, o_ref):
    acc = None
    for t in range(L):
        x = ps[t + 1][:, :32]
        r = jax.nn.relu(_dot(x, w1[...]) + b1[...])
        r = jax.nn.relu(_dot(r, w2[...]) + b2[...])
        ratio = _dot(r, w3[...]) + b3[...]
        term = ratio * mcap[t]
        acc = term if acc is None else acc + term
    o_ref[...] = acc


def _tc_readout(path_seq, mcap, w1, b1, w2, b2, w3, b3, block):
    full = lambda s: pl.BlockSpec(s, lambda i: (0, 0))
    return pl.pallas_call(
        _readout_body,
        grid=(P // block,),
        in_specs=[pl.BlockSpec((L + 1, block, 128), lambda i: (0, i, 0)),
                  pl.BlockSpec((L, block, 1), lambda i: (0, i, 0)),
                  full((32, 16)), full((1, 16)), full((16, 16)), full((1, 16)),
                  full((16, 1)), full((1, 1))],
        out_specs=pl.BlockSpec((block, 1), lambda i: (i, 0)),
        out_shape=jax.ShapeDtypeStruct((P, 1), F32),
    )(path_seq, mcap, w1, b1.reshape(1, -1), w2, b2.reshape(1, -1),
      w3, b3.reshape(1, -1))


@jax.jit
def _forward_impl(traffic, packets, eq_lambda, avg_pkts_lambda, exp_max_factor,
                  pkts_lambda_on, avg_t_off, avg_t_on, ar_a, sigma, capacity,
                  queue_size, weight, model, policy, priority, length,
                  queue_to_path, link_to_path, path_to_link, path_to_queue,
                  queue_to_link, pe_W1, pe_b1, pe_W2, pe_b2, le_W1, le_b1,
                  le_W2, le_b2, qe_W1, qe_b1, qe_W2, qe_b2, fw_Wx, fw_Wh,
                  fw_b, bw_Wx, bw_Wh, bw_b, qu_Wx, qu_Wh, qu_b, lu_Wx, lu_Wh,
                  lu_b, ro_W1, ro_b1, ro_W2, ro_b2, ro_W3, ro_b3):
    def zn(x, name):
        m, s = ZS[name]
        return (x - m) / s

    model_oh = jax.nn.one_hot(model, 7, dtype=F32)
    policy_oh = jax.nn.one_hot(policy, 4, dtype=F32)
    priority_oh = jax.nn.one_hot(priority, 3, dtype=F32)

    path_input = jnp.concatenate(
        [zn(traffic, 'traffic'), zn(packets, 'packets'), model_oh,
         zn(eq_lambda, 'eq_lambda'), zn(avg_pkts_lambda, 'avg_pkts_lambda'),
         zn(exp_max_factor, 'exp_max_factor'), zn(pkts_lambda_on, 'pkts_lambda_on'),
         zn(avg_t_off, 'avg_t_off'), zn(avg_t_on, 'avg_t_on'), ar_a, sigma],
        axis=1)
    queue_input = jnp.concatenate(
        [zn(queue_size, 'queue_size'), priority_oh, weight], axis=1)

    idx_q = queue_to_path.T.reshape(-1)
    idx_l = link_to_path.T.reshape(-1)
    idx_ps = (path_to_queue[:, :, 1] * P
              + path_to_queue[:, :, 0]).T.reshape(-1)
    idx_ql = queue_to_link.T.reshape(-1)
    idx_pl = path_to_link[:, :, 0].reshape(-1)

    wqa_p = jnp.concatenate([fw_Wx[:32], bw_Wx[:32]], axis=1)
    wla_p = jnp.concatenate([fw_Wx[32:], bw_Wx[32:]], axis=1)
    ba_p = jnp.concatenate([fw_b, bw_b])

    path_state = _tc_mlp2(path_input, pe_W1, pe_b1, pe_W2, pe_b2, 2000)
    h_fw = path_state[:, :16]
    h_bw = path_state[:, 16:]
    c_fw = jnp.zeros_like(h_fw)
    c_bw = jnp.zeros_like(h_bw)

    traffic128 = jnp.pad(traffic, ((0, 0), (0, 127)))
    capacity128 = jnp.pad(capacity, ((0, 0), (0, 127)))

    tr40 = _sc_gather(traffic128, idx_pl, pack=128).reshape(NL, 40)
    link_h, zl = _tc_link_embed(tr40, capacity, policy_oh, le_W1, le_b1,
                                le_W2, le_b2, wla_p, 400)
    queue_h, zq = _tc_queue_embed(queue_input, qe_W1, qe_b1, qe_W2, qe_b2,
                                  wqa_p, ba_p, 3000)
    queue_c = jnp.zeros_like(queue_h)
    link_c = jnp.zeros_like(link_h)

    cap_g = _sc_gather(capacity128, idx_l, pack=128).reshape(L, P, 1)
    mask = (jnp.arange(L)[:, None] < length[None, :]).astype(F32)
    mcap = mask[:, :, None] / cap_g

    path_seq = None
    for _ in range(8):
        gq = _sc_gather(zq, idx_q).reshape(L, P, 128)
        gl = _sc_gather(zl, idx_l).reshape(L, P, 128)
        path_seq, h_fw, c_fw, h_bw, c_bw = _tc_bilstm(
            gq, gl, h_fw, c_fw, h_bw, c_bw, fw_Wh, bw_Wh, 1000)
        gps = _sc_gather(path_seq.reshape((L + 1) * P, 128),
                         idx_ps).reshape(16, NQ, 128)
        queue_h, queue_c, zq, zql = _tc_queue_update(
            gps, queue_h, queue_c, qu_Wx, qu_Wh, qu_b, wqa_p, ba_p,
            lu_Wx, lu_b, 1000)
        gql = _sc_gather(zql, idx_ql).reshape(3, NL, 128)
        link_h, link_c, zl = _tc_link_update(
            gql, link_h, link_c, lu_Wh, wla_p, 2000)

    return _tc_readout(path_seq, mcap, ro_W1, ro_b1, ro_W2, ro_b2,
                       ro_W3, ro_b3, 2000)


def kernel(*args):
    return _forward_impl(*args)


# speedup vs baseline: 2.0116x; 1.0532x over previous
import functools

import jax
import jax.numpy as jnp
from jax.experimental import pallas as pl
from jax.experimental.pallas import tpu as pltpu
from jax.experimental.pallas import tpu_sc as plsc

P, L, NL, NQ = 50000, 8, 10000, 30000
NQP = 30720
F32 = jnp.float32
BF16 = jnp.bfloat16

ZS = {'traffic': (1385.4059, 859.8119), 'packets': (1.4015, 0.8933),
      'eq_lambda': (1350.9712, 858.3162), 'avg_pkts_lambda': (0.9117, 0.9724),
      'exp_max_factor': (6.6636, 4.7151), 'pkts_lambda_on': (0.9116, 1.6513),
      'avg_t_off': (1.6649, 2.3564), 'avg_t_on': (1.6649, 2.3564),
      'ar_a': (0.0, 1.0), 'sigma': (0.0, 1.0),
      'capacity': (27611.0918, 20090.6211), 'queue_size': (30259.1055, 21410.0957)}


_W = 128


def _sc_gather(table, idx, pack=1):
    m0 = idx.shape[0]
    d = table.shape[1]
    m = ((m0 + _W - 1) // _W) * _W
    npack = 16 if pack == 'sum16' else pack
    if npack > 1:
        assert m == m0
    if m != m0:
        idx = jnp.concatenate([idx, jnp.zeros((m - m0,), jnp.int32)])
    idx2 = idx.reshape(m // _W, _W)
    mesh = plsc.VectorSubcoreMesh(core_axis_name="core", subcore_axis_name="subcore")
    scratch = [pltpu.VMEM((_W, d), table.dtype)] if npack > 1 else []

    @functools.partial(
        pl.kernel,
        out_type=jax.ShapeDtypeStruct((m // npack, d), table.dtype),
        mesh=mesh,
        scratch_types=scratch,
        compiler_params=pltpu.CompilerParams(use_tc_tiling_on_sc=False,
                                             needs_layout_passes=False),
    )
    def gk(x_hbm, i_hbm, o_hbm, *maybe_buf):
        def body(i_vmem, o_vmem):
            if pack == 1:
                pltpu.sync_copy(x_hbm.at[i_vmem.at[0]], o_vmem)
            elif pack == 'sum16':
                buf = maybe_buf[0]
                pltpu.sync_copy(x_hbm.at[i_vmem.at[0]], buf)
                for q in range(_W // 16):
                    a0 = buf[16 * q, pl.ds(0, 16)]
                    a1 = buf[16 * q, pl.ds(16, 16)]
                    for r in range(1, 16):
                        a0 = a0 + buf[16 * q + r, pl.ds(0, 16)]
                        a1 = a1 + buf[16 * q + r, pl.ds(16, 16)]
                    o_vmem[q, pl.ds(0, 16)] = a0
                    o_vmem[q, pl.ds(16, 16)] = a1
            elif pack == 4:
                buf = maybe_buf[0]
                pltpu.sync_copy(x_hbm.at[i_vmem.at[0]], buf)

                @pl.loop(0, _W // 4)
                def _(k):
                    for c in range(4):
                        for h in range(2):
                            o_vmem[k, pl.ds(32 * c + 16 * h, 16)] = (
                                buf[4 * k + c, pl.ds(16 * h, 16)])
            else:
                buf = maybe_buf[0]
                pltpu.sync_copy(x_hbm.at[i_vmem.at[0]], buf)
                cols = jnp.zeros((16,), jnp.int32)
                for j in range(8):
                    rows = jax.lax.iota(jnp.int32, 16) + 16 * j
                    o_vmem[0, pl.ds(16 * j, 16)] = plsc.load_gather(
                        buf, [rows, cols])

        pltpu.emit_pipeline(
            body,
            grid=(m // _W,),
            in_specs=[pl.BlockSpec((1, _W), lambda i: (i, 0))],
            out_specs=[pl.BlockSpec((_W // npack, d), lambda i: (i, 0))],
            core_axis_name=("core", "subcore"),
            dimension_semantics=(pltpu.PARALLEL,),
        )(i_hbm, o_hbm)

    out = gk(table, idx2)
    return out[:m0] if m != m0 else out


def _dot(a, b):
    return jnp.dot(a, b, preferred_element_type=F32)


def _sig(x):
    return jax.nn.sigmoid(x)


def _mlp2_body(x_ref, w1, b1, w2, b2, o_ref):
    h = jax.nn.relu(_dot(x_ref[...], w1[...]) + b1[...])
    o_ref[...] = jax.nn.relu(_dot(h, w2[...]) + b2[...])


def _tc_mlp2(x, w1, b1, w2, b2, block):
    n, f = x.shape
    dh, do = w1.shape[1], w2.shape[1]
    full = lambda s: pl.BlockSpec(s, lambda i: (0, 0))
    return pl.pallas_call(
        _mlp2_body,
        grid=(n // block,),
        in_specs=[pl.BlockSpec((block, f), lambda i: (i, 0)),
                  full((f, dh)), full((1, dh)), full((dh, do)), full((1, do))],
        out_specs=pl.BlockSpec((block, do), lambda i: (i, 0)),
        out_shape=jax.ShapeDtypeStruct((n, do), F32),
    )(x, w1, b1.reshape(1, -1), w2, b2.reshape(1, -1))


def _link_embed_body(tr, cap, pol, w1, b1, w2, b2, wla, h_ref, zl_ref):
    tsum = jnp.sum(tr[...], axis=1, keepdims=True)
    load = tsum / cap[...]
    x = jnp.concatenate([load, pol[...]], axis=1)
    h = jax.nn.relu(_dot(x, w1[...]) + b1[...])
    h = jax.nn.relu(_dot(h, w2[...]) + b2[...])
    h_ref[...] = h
    zl_ref[...] = _dot(h, wla[...])


def _tc_link_embed(tr40, cap, pol, w1, b1, w2, b2, wla, block):
    full = lambda s: pl.BlockSpec(s, lambda i: (0, 0))
    return pl.pallas_call(
        _link_embed_body,
        grid=(NL // block,),
        in_specs=[pl.BlockSpec((block, 40), lambda i: (i, 0)),
                  pl.BlockSpec((block, 1), lambda i: (i, 0)),
                  pl.BlockSpec((block, 4), lambda i: (i, 0)),
                  full((5, 32)), full((1, 32)), full((32, 32)), full((1, 32)),
                  full((32, 128))],
        out_specs=[pl.BlockSpec((block, 32), lambda i: (i, 0)),
                   pl.BlockSpec((block, 128), lambda i: (i, 0))],
        out_shape=[jax.ShapeDtypeStruct((NL, 32), F32),
                   jax.ShapeDtypeStruct((NL, 128), F32)],
    )(tr40, cap, pol, w1, b1.reshape(1, -1), w2, b2.reshape(1, -1), wla)


def _queue_embed_body(x_ref, w1, b1, w2, b2, wqa, ba, h_ref, zq_ref):
    h = jax.nn.relu(_dot(x_ref[...], w1[...]) + b1[...])
    h = jax.nn.relu(_dot(h, w2[...]) + b2[...])
    h_ref[...] = h
    zq_ref[...] = _dot(h, wqa[...]) + ba[...]


def _tc_queue_embed(x, w1, b1, w2, b2, wqa, ba, block):
    full = lambda s: pl.BlockSpec(s, lambda i: (0, 0))
    return pl.pallas_call(
        _queue_embed_body,
        grid=(NQ // block,),
        in_specs=[pl.BlockSpec((block, 5), lambda i: (i, 0)),
                  full((5, 32)), full((1, 32)), full((32, 32)), full((1, 32)),
                  full((32, 128)), full((1, 128))],
        out_specs=[pl.BlockSpec((block, 32), lambda i: (i, 0)),
                   pl.BlockSpec((block, 128), lambda i: (i, 0))],
        out_shape=[jax.ShapeDtypeStruct((NQ, 32), F32),
                   jax.ShapeDtypeStruct((NQ, 128), F32)],
    )(x, w1, b1.reshape(1, -1), w2, b2.reshape(1, -1), wqa, ba.reshape(1, -1))


def _bilstm_body(gq, gl, hf0, cf0, hb0, cb0, whf, whb,
                 ps_ref, hf_ref, cf_ref, hb_ref, cb_ref):
    z = gq[...] + gl[...]
    zf = z[:, :, :64]
    zb = z[:, :, 64:]
    hf = hf0[...]
    cf = cf0[...]
    sf = []
    for t in range(L):
        zt = zf[t] + _dot(hf, whf[...])
        i_, f_, g_, o_ = zt[:, :16], zt[:, 16:32], zt[:, 32:48], zt[:, 48:64]
        cf = _sig(f_) * cf + _sig(i_) * jnp.tanh(g_)
        hf = _sig(o_) * jnp.tanh(cf)
        sf.append(hf)
    hb = hb0[...]
    cb = cb0[...]
    sb = [None] * L
    for t in range(L - 1, -1, -1):
        zt = zb[t] + _dot(hb, whb[...])
        i_, f_, g_, o_ = zt[:, :16], zt[:, 16:32], zt[:, 32:48], zt[:, 48:64]
        cb = _sig(f_) * cb + _sig(i_) * jnp.tanh(g_)
        hb = _sig(o_) * jnp.tanh(cb)
        sb[t] = hb
    zpad = jnp.zeros((hf.shape[0], 96), F32)
    ps_ref[0] = jnp.concatenate([hf0[...], hb0[...], zpad], axis=1)
    for t in range(L):
        ps_ref[t + 1] = jnp.concatenate([sf[t], sb[t], zpad], axis=1)
    hf_ref[...] = hf
    cf_ref[...] = cf
    hb_ref[...] = hb
    cb_ref[...] = cb


def _tc_bilstm(gq, gl, hf, cf, hb, cb, whf, whb, block):
    full = lambda s: pl.BlockSpec(s, lambda i: (0, 0))
    st = lambda: pl.BlockSpec((block, 16), lambda i: (i, 0))
    return pl.pallas_call(
        _bilstm_body,
        grid=(P // block,),
        in_specs=[pl.BlockSpec((L, block, 128), lambda i: (0, i, 0)),
                  pl.BlockSpec((L, block, 128), lambda i: (0, i, 0)),
                  st(), st(), st(), st(),
                  full((16, 64)), full((16, 64))],
        out_specs=[pl.BlockSpec((L + 1, block, 128), lambda i: (0, i, 0)),
                   st(), st(), st(), st()],
        out_shape=[jax.ShapeDtypeStruct((L + 1, P, 128), F32),
                   jax.ShapeDtypeStruct((P, 16), F32),
                   jax.ShapeDtypeStruct((P, 16), F32),
                   jax.ShapeDtypeStruct((P, 16), F32),
                   jax.ShapeDtypeStruct((P, 16), F32)],
    )(gq, gl, hf, cf, hb, cb, whf, whb)


def _queue_update_body(gps, h0, c0, wx, wh, b, wqa, ba, wzl, bzl,
                       h_ref, c_ref, zq_ref, zql_ref):
    ps = gps[...][:, :32]
    z = _dot(ps, wx[...]) + _dot(h0[...], wh[...]) + b[...]
    i_, f_, g_, o_ = z[:, :32], z[:, 32:64], z[:, 64:96], z[:, 96:128]
    c = _sig(f_) * c0[...] + _sig(i_) * jnp.tanh(g_)
    h = _sig(o_) * jnp.tanh(c)
    h_ref[...] = h
    c_ref[...] = c
    zq_ref[...] = _dot(h, wqa[...]) + ba[...]
    zql_ref[...] = _dot(h, wzl[...]) + bzl[...]


def _tc_queue_update(gps, h, c, wx, wh, b, wqa, ba, wzl, bzl, block):
    full = lambda s: pl.BlockSpec(s, lambda i: (0, 0))
    s32 = lambda: pl.BlockSpec((block, 32), lambda i: (i, 0))
    s128 = lambda: pl.BlockSpec((block, 128), lambda i: (i, 0))
    return pl.pallas_call(
        _queue_update_body,
        grid=(NQ // block,),
        in_specs=[pl.BlockSpec((block, 128), lambda i: (i, 0)),
                  s32(), s32(),
                  full((32, 128)), full((32, 128)), full((1, 128)),
                  full((32, 128)), full((1, 128)),
                  full((32, 128)), full((1, 128))],
        out_specs=[s32(), s32(), s128(), s128()],
        out_shape=[jax.ShapeDtypeStruct((NQ, 32), F32),
                   jax.ShapeDtypeStruct((NQ, 32), F32),
                   jax.ShapeDtypeStruct((NQ, 128), F32),
                   jax.ShapeDtypeStruct((NQ, 128), F32)],
    )(gps, h, c, wx, wh, b.reshape(1, -1), wqa, ba.reshape(1, -1),
      wzl, bzl.reshape(1, -1))


def _link_update_body(gql, h0, c0, wh, wla, h_ref, c_ref, zl_ref):
    g = gql[...]
    h = h0[...]
    c = c0[...]
    for t in range(3):
        z = g[t] + _dot(h, wh[...])
        i_, f_, g_, o_ = z[:, :32], z[:, 32:64], z[:, 64:96], z[:, 96:128]
        c = _sig(f_) * c + _sig(i_) * jnp.tanh(g_)
        h = _sig(o_) * jnp.tanh(c)
    h_ref[...] = h
    c_ref[...] = c
    zl_ref[...] = _dot(h, wla[...])


def _tc_link_update(gql, h, c, wh, wla, block):
    full = lambda s: pl.BlockSpec(s, lambda i: (0, 0))
    s32 = lambda: pl.BlockSpec((block, 32), lambda i: (i, 0))
    return pl.pallas_call(
        _link_update_body,
        grid=(NL // block,),
        in_specs=[pl.BlockSpec((3, block, 128), lambda i: (0, i, 0)),
                  s32(), s32(),
                  full((32, 128)), full((32, 128))],
        out_specs=[s32(), s32(),
                   pl.BlockSpec((block, 128), lambda i: (i, 0))],
        out_shape=[jax.ShapeDtypeStruct((NL, 32), F32),
                   jax.ShapeDtypeStruct((NL, 32), F32),
                   jax.ShapeDtypeStruct((NL, 128), F32)],
    )(gql, h, c, wh, wla)


def _readout_body(ps, mcap, w1, b1, w2, b2, w3, b3, o_ref):
    acc = None
    for t in range(L):
        x = ps[t + 1][:, :32]
        r = jax.nn.relu(_dot(x, w1[...]) + b1[...])
        r = jax.nn.relu(_dot(r, w2[...]) + b2[...])
        ratio = _dot(r, w3[...]) + b3[...]
        term = ratio * mcap[t]
        acc = term if acc is None else acc + term
    o_ref[...] = acc


def _tc_readout(path_seq, mcap, w1, b1, w2, b2, w3, b3, block):
    full = lambda s: pl.BlockSpec(s, lambda i: (0, 0))
    return pl.pallas_call(
        _readout_body,
        grid=(P // block,),
        in_specs=[pl.BlockSpec((L + 1, block, 128), lambda i: (0, i, 0)),
                  pl.BlockSpec((L, block, 1), lambda i: (0, i, 0)),
                  full((32, 16)), full((1, 16)), full((16, 16)), full((1, 16)),
                  full((16, 1)), full((1, 1))],
        out_specs=pl.BlockSpec((block, 1), lambda i: (i, 0)),
        out_shape=jax.ShapeDtypeStruct((P, 1), F32),
    )(path_seq, mcap, w1, b1.reshape(1, -1), w2, b2.reshape(1, -1),
      w3, b3.reshape(1, -1))


@jax.jit
def _forward_impl(traffic, packets, eq_lambda, avg_pkts_lambda, exp_max_factor,
                  pkts_lambda_on, avg_t_off, avg_t_on, ar_a, sigma, capacity,
                  queue_size, weight, model, policy, priority, length,
                  queue_to_path, link_to_path, path_to_link, path_to_queue,
                  queue_to_link, pe_W1, pe_b1, pe_W2, pe_b2, le_W1, le_b1,
                  le_W2, le_b2, qe_W1, qe_b1, qe_W2, qe_b2, fw_Wx, fw_Wh,
                  fw_b, bw_Wx, bw_Wh, bw_b, qu_Wx, qu_Wh, qu_b, lu_Wx, lu_Wh,
                  lu_b, ro_W1, ro_b1, ro_W2, ro_b2, ro_W3, ro_b3):
    def zn(x, name):
        m, s = ZS[name]
        return (x - m) / s

    model_oh = jax.nn.one_hot(model, 7, dtype=F32)
    policy_oh = jax.nn.one_hot(policy, 4, dtype=F32)
    priority_oh = jax.nn.one_hot(priority, 3, dtype=F32)

    path_input = jnp.concatenate(
        [zn(traffic, 'traffic'), zn(packets, 'packets'), model_oh,
         zn(eq_lambda, 'eq_lambda'), zn(avg_pkts_lambda, 'avg_pkts_lambda'),
         zn(exp_max_factor, 'exp_max_factor'), zn(pkts_lambda_on, 'pkts_lambda_on'),
         zn(avg_t_off, 'avg_t_off'), zn(avg_t_on, 'avg_t_on'), ar_a, sigma],
        axis=1)
    queue_input = jnp.concatenate(
        [zn(queue_size, 'queue_size'), priority_oh, weight], axis=1)

    idx_q = queue_to_path.T.reshape(-1)
    idx_l = link_to_path.T.reshape(-1)
    idx_ps = (path_to_queue[:, :, 1] * P
              + path_to_queue[:, :, 0]).reshape(-1)
    idx_ql = queue_to_link.T.reshape(-1)
    idx_pl = path_to_link[:, :, 0].reshape(-1)

    wqa_p = jnp.concatenate([fw_Wx[:32], bw_Wx[:32]], axis=1)
    wla_p = jnp.concatenate([fw_Wx[32:], bw_Wx[32:]], axis=1)
    ba_p = jnp.concatenate([fw_b, bw_b])

    path_state = _tc_mlp2(path_input, pe_W1, pe_b1, pe_W2, pe_b2, 2000)
    h_fw = path_state[:, :16]
    h_bw = path_state[:, 16:]
    c_fw = jnp.zeros_like(h_fw)
    c_bw = jnp.zeros_like(h_bw)

    traffic128 = jnp.pad(traffic, ((0, 0), (0, 127)))
    capacity128 = jnp.pad(capacity, ((0, 0), (0, 127)))

    tr40 = _sc_gather(traffic128, idx_pl, pack=128).reshape(NL, 40)
    link_h, zl = _tc_link_embed(tr40, capacity, policy_oh, le_W1, le_b1,
                                le_W2, le_b2, wla_p, 400)
    queue_h, zq = _tc_queue_embed(queue_input, qe_W1, qe_b1, qe_W2, qe_b2,
                                  wqa_p, ba_p, 3000)
    queue_c = jnp.zeros_like(queue_h)
    link_c = jnp.zeros_like(link_h)

    cap_g = _sc_gather(capacity128, idx_l, pack=128).reshape(L, P, 1)
    mask = (jnp.arange(L)[:, None] < length[None, :]).astype(F32)
    mcap = mask[:, :, None] / cap_g

    path_seq = None
    for _ in range(8):
        gq = _sc_gather(zq, idx_q).reshape(L, P, 128)
        gl = _sc_gather(zl, idx_l).reshape(L, P, 128)
        path_seq, h_fw, c_fw, h_bw, c_bw = _tc_bilstm(
            gq, gl, h_fw, c_fw, h_bw, c_bw, fw_Wh, bw_Wh, 1000)
        gps = _sc_gather(path_seq.reshape((L + 1) * P, 128),
                         idx_ps, pack='sum16')
        queue_h, queue_c, zq, zql = _tc_queue_update(
            gps, queue_h, queue_c, qu_Wx, qu_Wh, qu_b, wqa_p, ba_p,
            lu_Wx, lu_b, 1000)
        gql = _sc_gather(zql, idx_ql).reshape(3, NL, 128)
        link_h, link_c, zl = _tc_link_update(
            gql, link_h, link_c, lu_Wh, wla_p, 2000)

    return _tc_readout(path_seq, mcap, ro_W1, ro_b1, ro_W2, ro_b2,
                       ro_W3, ro_b3, 2000)


def kernel(*args):
    return _forward_impl(*args)
